# baseline (speedup 1.0000x reference)
import jax
import jax.numpy as jnp
from jax import lax
from jax.experimental import pallas as pl
from jax.experimental.pallas import tpu as pltpu
from jax.experimental.pallas import tpu_sc as plsc

DIM = 64
BATCH = 16384
N_ROWS = 1000000
NC = 2
NS = 16
NW = NC * NS
LANES = 16
EV_ENT = 3 * BATCH
EV_REL = BATCH
ENT_PW = EV_ENT // NW
REL_PW = EV_REL // NW
WIN = 248
STG = 128
B_PER_W = BATCH // NW
CHUNK = 128
NCHUNK = B_PER_W // CHUNK


def _extract(vec, q):
    return jnp.squeeze(lax.slice(vec, (q,), (q + 1,)))


def _copy_events(tab, rows_v, stag_v, win_v, out, n_ev, out_base):
    gb = STG // LANES

    def grp(g, cur_start):
        vec = rows_v[pl.ds(pl.multiple_of(g * LANES, LANES), LANES)]
        for q in range(LANES):
            row = _extract(vec, q)
            newstart = pl.multiple_of(
                jnp.minimum(row & ~7, N_ROWS - WIN), 8)
            need = (row - cur_start) >= WIN

            @pl.when(need)
            def _():
                pltpu.sync_copy(tab.at[pl.ds(newstart, WIN)], win_v)

            cur_start = jnp.where(need, newstart, cur_start)
            rr = row - cur_start
            slot = (g & (2 * gb - 1)) * LANES + q
            for k in range(DIM // LANES):
                sl = pl.ds(k * LANES, LANES)
                stag_v[slot, sl] = win_v[rr, sl]

        @pl.when((g & (gb - 1)) == (gb - 1))
        def _():
            half = pl.multiple_of(((g - (gb - 1)) & (2 * gb - 1)) * LANES,
                                  STG)
            dst = pl.multiple_of(out_base + (g - (gb - 1)) * LANES, STG)
            pltpu.sync_copy(stag_v.at[pl.ds(half, STG)],
                            out.at[pl.ds(dst, STG)])

        return cur_start

    lax.fori_loop(0, n_ev // LANES, grp, jnp.int32(-8 * WIN))


def _scan_body(ent, rel, ent_rows, rel_rows, g1, g2,
               erows_v, rrows_v, stag_v, win_v, sem):
    wid = lax.axis_index("s") * NC + lax.axis_index("c")
    pltpu.sync_copy(ent_rows.at[pl.ds(wid * ENT_PW, ENT_PW)], erows_v)
    pltpu.sync_copy(rel_rows.at[pl.ds(wid * REL_PW, REL_PW)], rrows_v)
    _copy_events(ent, erows_v, stag_v, win_v, g1, ENT_PW, wid * ENT_PW)
    _copy_events(rel, rrows_v, stag_v, win_v, g2, REL_PW, wid * REL_PW)


@jax.jit
def _scan_sc(ent, rel, ent_rows, rel_rows):
    mesh = plsc.VectorSubcoreMesh(core_axis_name="c", subcore_axis_name="s")
    k = pl.kernel(
        _scan_body,
        out_type=(jax.ShapeDtypeStruct((EV_ENT, DIM), jnp.float32),
                  jax.ShapeDtypeStruct((EV_REL, DIM), jnp.float32)),
        mesh=mesh,
        scratch_types=[
            pltpu.VMEM((ENT_PW,), jnp.int32),
            pltpu.VMEM((REL_PW,), jnp.int32),
            pltpu.VMEM((2 * STG, DIM), jnp.float32),
            pltpu.VMEM((WIN, DIM), jnp.float32),
            pltpu.SemaphoreType.DMA,
        ],
    )
    return k(ent, rel, ent_rows, rel_rows)


def _gather_body(g1, g2, h_p, r_p, tp_p, tn_p, out,
                 idx_v, h_v, r_v, tp_v, tn_v, acc_v, sem):
    wid = lax.axis_index("s") * NC + lax.axis_index("c")
    base = wid * B_PER_W

    acc = jnp.zeros((LANES,), jnp.float32)
    for c in range(NCHUNK):
        off = base + c * CHUNK
        pltpu.sync_copy(h_p.at[pl.ds(off, CHUNK)], idx_v.at[0])
        pltpu.sync_copy(r_p.at[pl.ds(off, CHUNK)], idx_v.at[1])
        pltpu.sync_copy(tp_p.at[pl.ds(off, CHUNK)], idx_v.at[2])
        pltpu.sync_copy(tn_p.at[pl.ds(off, CHUNK)], idx_v.at[3])
        cp_h = pltpu.make_async_copy(g1.at[idx_v.at[0]], h_v, sem)
        cp_r = pltpu.make_async_copy(g2.at[idx_v.at[1]], r_v, sem)
        cp_tp = pltpu.make_async_copy(g1.at[idx_v.at[2]], tp_v, sem)
        cp_tn = pltpu.make_async_copy(g1.at[idx_v.at[3]], tn_v, sem)
        cp_h.start(); cp_r.start(); cp_tp.start(); cp_tn.start()
        cp_h.wait(); cp_r.wait(); cp_tp.wait(); cp_tn.wait()

        def row(j, a):
            for q in range(DIM // LANES):
                sl = pl.ds(q * LANES, LANES)
                s = h_v[j, sl] + r_v[j, sl]
                dp = s - tp_v[j, sl]
                dn = s - tn_v[j, sl]
                a = a + (dn * dn - dp * dp)
            return a

        acc = lax.fori_loop(0, CHUNK, row, acc)

    acc_v[...] = acc
    pltpu.sync_copy(acc_v, out.at[pl.ds(wid * LANES, LANES)])


@jax.jit
def _gather_sc(g1, g2, h_p, r_p, tp_p, tn_p):
    mesh = plsc.VectorSubcoreMesh(core_axis_name="c", subcore_axis_name="s")
    k = pl.kernel(
        _gather_body,
        out_type=jax.ShapeDtypeStruct((NW * LANES,), jnp.float32),
        mesh=mesh,
        scratch_types=[
            pltpu.VMEM((4, CHUNK), jnp.int32),
            pltpu.VMEM((CHUNK, DIM), jnp.float32),
            pltpu.VMEM((CHUNK, DIM), jnp.float32),
            pltpu.VMEM((CHUNK, DIM), jnp.float32),
            pltpu.VMEM((CHUNK, DIM), jnp.float32),
            pltpu.VMEM((LANES,), jnp.float32),
            pltpu.SemaphoreType.DMA,
        ],
        compiler_params=pltpu.CompilerParams(use_tc_tiling_on_sc=False),
    )
    return k(g1, g2, h_p, r_p, tp_p, tn_p)


def kernel(data, entity_embedding_matrix, relation_embedding_matrix):
    idx = data.astype(jnp.int32)
    ent_vals = jnp.concatenate([idx[:, 0], idx[:, 2], idx[:, 3]])
    rel_vals = idx[:, 1]
    ent_order = jnp.argsort(ent_vals)
    rel_order = jnp.argsort(rel_vals)
    ent_rows = ent_vals[ent_order]
    rel_rows = rel_vals[rel_order]
    ar_e = jnp.arange(EV_ENT, dtype=jnp.int32)
    ar_r = jnp.arange(EV_REL, dtype=jnp.int32)
    ent_inv = jnp.zeros((EV_ENT,), jnp.int32).at[ent_order].set(ar_e)
    rel_inv = jnp.zeros((EV_REL,), jnp.int32).at[rel_order].set(ar_r)
    h_p = ent_inv[:BATCH]
    tp_p = ent_inv[BATCH:2 * BATCH]
    tn_p = ent_inv[2 * BATCH:]
    r_p = rel_inv

    g1, g2 = _scan_sc(entity_embedding_matrix, relation_embedding_matrix,
                      ent_rows, rel_rows)
    partials = _gather_sc(g1, g2, h_p, r_p, tp_p, tn_p)
    return jnp.sum(partials)


# speedup vs baseline: 1.7785x; 1.7785x over previous
import jax
import jax.numpy as jnp
from jax import lax
from jax.experimental import pallas as pl
from jax.experimental.pallas import tpu as pltpu
from jax.experimental.pallas import tpu_sc as plsc

DIM = 64
BATCH = 16384
NC = 2
NS = 16
NW = NC * NS
LANES = 16
B_PER_W = BATCH // NW
G = 32
NCH = B_PER_W // G
NSTREAM = 4


def _extract(vec, q):
    return jnp.squeeze(lax.slice(vec, (q,), (q + 1,)))


def _transe_body(ent, rel, comb, out, idx_v, buf_v, acc_v, sem):
    wid = lax.axis_index("s") * NC + lax.axis_index("c")

    pltpu.sync_copy(comb.at[wid], idx_v)

    def chunk(c, acc):
        base = c * (NSTREAM * G)
        for s, tab in enumerate((ent, rel, ent, ent)):
            for sg in range(G // LANES):
                iv = idx_v[pl.ds(base + s * G + sg * LANES, LANES)]
                for q in range(LANES):
                    pltpu.make_async_copy(
                        tab.at[pl.ds(_extract(iv, q), 1)],
                        buf_v.at[pl.ds(s * G + sg * LANES + q, 1)], sem).start()
        pltpu.make_async_copy(ent.at[pl.ds(0, NSTREAM * G)], buf_v, sem).wait()
        for q in range(G):
            for k in range(DIM // LANES):
                sl = pl.ds(k * LANES, LANES)
                s_ = buf_v[q, sl] + buf_v[G + q, sl]
                dp = s_ - buf_v[2 * G + q, sl]
                dn = s_ - buf_v[3 * G + q, sl]
                acc = acc + (dn * dn - dp * dp)
        return acc

    acc = lax.fori_loop(0, NCH, chunk, jnp.zeros((LANES,), jnp.float32))
    acc_v[...] = acc
    pltpu.sync_copy(acc_v, out.at[pl.ds(wid * LANES, LANES)])


@jax.jit
def _transe_sc(ent, rel, comb):
    mesh = plsc.VectorSubcoreMesh(core_axis_name="c", subcore_axis_name="s")
    grid_kernel = pl.kernel(
        _transe_body,
        out_type=jax.ShapeDtypeStruct((NW * LANES,), jnp.float32),
        mesh=mesh,
        scratch_types=[
            pltpu.VMEM((NCH * NSTREAM * G,), jnp.int32),
            pltpu.VMEM((NSTREAM * G, DIM), jnp.float32),
            pltpu.VMEM((LANES,), jnp.float32),
            pltpu.SemaphoreType.DMA,
        ],
    )
    return grid_kernel(ent, rel, comb)


def kernel(data, entity_embedding_matrix, relation_embedding_matrix):
    idx = data.astype(jnp.int32)
    comb = (idx.reshape(NW, NCH, G, NSTREAM)
               .transpose(0, 1, 3, 2)
               .reshape(NW, NCH * NSTREAM * G))
    partials = _transe_sc(entity_embedding_matrix, relation_embedding_matrix,
                          comb)
    return jnp.sum(partials)
